# baseline (speedup 1.0000x reference)
import jax
import jax.numpy as jnp
from jax.experimental import pallas as pl
from jax.experimental.pallas import tpu as pltpu

_REPS = 4


def _st_embed_kernel(wv_ref, wtf_ref, out_ref, m_ref, sem):
    N, D = wv_ref.shape
    TD = wtf_ref.shape[1]
    T = TD // D
    m = jnp.tile(wv_ref[...], (1, T)) + wtf_ref[...]
    for r in range(_REPS):
        m_ref[pl.ds(r * N, N), :] = m
    B = out_ref.shape[0] // N
    n_dma = B // _REPS
    for i in range(n_dma):
        pltpu.make_async_copy(
            m_ref, out_ref.at[pl.ds(i * N * _REPS, N * _REPS), :], sem
        ).start()
    for i in range(n_dma):
        pltpu.make_async_copy(
            m_ref, out_ref.at[pl.ds(i * N * _REPS, N * _REPS), :], sem
        ).wait()


def kernel(x, W_veh, W_time, W_pos):
    B, N, T, F = x.shape
    D = W_veh.shape[1]
    out2 = pl.pallas_call(
        _st_embed_kernel,
        in_specs=[
            pl.BlockSpec(memory_space=pltpu.VMEM),
            pl.BlockSpec(memory_space=pltpu.VMEM),
        ],
        out_specs=pl.BlockSpec(memory_space=pl.ANY),
        out_shape=jax.ShapeDtypeStruct((B * N, T * D), W_veh.dtype),
        scratch_shapes=[
            pltpu.VMEM((_REPS * N, T * D), W_veh.dtype),
            pltpu.SemaphoreType.DMA,
        ],
    )(W_veh[:N], W_time[:T].reshape(1, T * D))
    return out2.reshape(B, N, T, D)


# speedup vs baseline: 1.3266x; 1.3266x over previous
import jax
import jax.numpy as jnp
from jax.experimental import pallas as pl
from jax.experimental.pallas import tpu as pltpu

_REPS = 4


def _st_embed_kernel(wv_ref, wt_ref, out_ref, m_ref, sem):
    N, D = wv_ref.shape
    T = wt_ref.shape[0]
    m3 = wv_ref[...][:, None, :] + wt_ref[...][None, :, :]
    for r in range(_REPS):
        m_ref[r] = m3
    B = out_ref.shape[0]
    n_dma = B // _REPS
    for i in range(n_dma):
        pltpu.make_async_copy(
            m_ref, out_ref.at[pl.ds(i * _REPS, _REPS)], sem
        ).start()
    for i in range(n_dma):
        pltpu.make_async_copy(
            m_ref, out_ref.at[pl.ds(i * _REPS, _REPS)], sem
        ).wait()


def kernel(x, W_veh, W_time, W_pos):
    B, N, T, F = x.shape
    D = W_veh.shape[1]
    return pl.pallas_call(
        _st_embed_kernel,
        in_specs=[
            pl.BlockSpec(memory_space=pltpu.VMEM),
            pl.BlockSpec(memory_space=pltpu.VMEM),
        ],
        out_specs=pl.BlockSpec(memory_space=pl.ANY),
        out_shape=jax.ShapeDtypeStruct((B, N, T, D), W_veh.dtype),
        scratch_shapes=[
            pltpu.VMEM((_REPS, N, T, D), W_veh.dtype),
            pltpu.SemaphoreType.DMA,
        ],
    )(W_veh[:N], W_time[:T])


# speedup vs baseline: 5.9925x; 4.5173x over previous
import jax
import jax.numpy as jnp
from jax.experimental import pallas as pl


def _st_embed_kernel(wv_ref, wt_ref, out_ref):
    T, D = wt_ref.shape
    B = out_ref.shape[3]
    n = pl.program_id(0)
    m = wv_ref[pl.ds(n, 1), :] + wt_ref[...]
    out_ref[0] = jnp.broadcast_to(m[:, :, None], (T, D, B))


def kernel(x, W_veh, W_time, W_pos):
    B, N, T, F = x.shape
    D = W_veh.shape[1]
    out = pl.pallas_call(
        _st_embed_kernel,
        grid=(N,),
        in_specs=[
            pl.BlockSpec((N, D), lambda i: (0, 0)),
            pl.BlockSpec((T, D), lambda i: (0, 0)),
        ],
        out_specs=pl.BlockSpec((1, T, D, B), lambda i: (i, 0, 0, 0)),
        out_shape=jax.ShapeDtypeStruct((N, T, D, B), W_veh.dtype),
    )(W_veh[:N], W_time[:T])
    return jnp.transpose(out, (3, 0, 1, 2))


# speedup vs baseline: 7.3214x; 1.2218x over previous
import jax
import jax.numpy as jnp
from jax.experimental import pallas as pl
from jax.experimental.pallas import tpu as pltpu


_BN = 2


def _st_embed_kernel(wv_ref, wt_ref, out_ref):
    T, D = wt_ref.shape
    B = out_ref.shape[3]
    i = pl.program_id(0)
    for j in range(_BN):
        m = wv_ref[pl.ds(i * _BN + j, 1), :] + wt_ref[...]
        out_ref[j] = jnp.broadcast_to(m[:, :, None], (T, D, B))


def kernel(x, W_veh, W_time, W_pos):
    B, N, T, F = x.shape
    D = W_veh.shape[1]
    out = pl.pallas_call(
        _st_embed_kernel,
        grid=(N // _BN,),
        in_specs=[
            pl.BlockSpec((N, D), lambda i: (0, 0)),
            pl.BlockSpec((T, D), lambda i: (0, 0)),
        ],
        out_specs=pl.BlockSpec((_BN, T, D, B), lambda i: (i, 0, 0, 0)),
        out_shape=jax.ShapeDtypeStruct((N, T, D, B), W_veh.dtype),
        compiler_params=pltpu.CompilerParams(
            dimension_semantics=("parallel",),
        ),
    )(W_veh[:N], W_time[:T])
    return jnp.transpose(out, (3, 0, 1, 2))
